# baseline (speedup 1.0000x reference)
import functools

import jax
import jax.numpy as jnp
from jax import lax
from jax.experimental import pallas as pl
from jax.experimental.pallas import tpu as pltpu
from jax.experimental.pallas import tpu_sc as plsc

_G = 256
_N = 100000
_D = 128
_NC = 2
_NS = 16
_NW = _NC * _NS

_TC_BLK = 2048
_N_TC = 16 * _TC_BLK
_N_SC = _N - _N_TC

_BLK = 128
_SC_NBLKS = (_N_SC + _BLK - 1) // _BLK
_BASE_BLKS = _SC_NBLKS // _NW
_EXTRA = _SC_NBLKS - _BASE_BLKS * _NW
_MAX_BLKS = _BASE_BLKS + (1 if _EXTRA else 0)
_ZROWS = _G // _NS


def _sc_segment_sum(x, batch):
    mesh = plsc.VectorSubcoreMesh(core_axis_name="c", subcore_axis_name="s")

    @functools.partial(
        pl.kernel,
        out_type=jax.ShapeDtypeStruct((_NC, _G, _D), jnp.float32),
        mesh=mesh,
        scratch_types=[
            pltpu.VMEM((2, _BLK, _D), jnp.float32),
            pltpu.VMEM((2, _BLK), jnp.int32),
            pltpu.VMEM((_ZROWS, _D), jnp.float32),
            pltpu.VMEM_SHARED((_G + 8, _D), jnp.float32),
            pltpu.SemaphoreType.DMA((2,)),
            pltpu.SemaphoreType.DMA((2,)),
            pltpu.SemaphoreType.DMA((2,)),
        ],
    )
    def seg_sum(x_hbm, b_hbm, out_hbm, xbuf, idx2, zbuf, acc, gsem, isem,
                ssem):
        cid = lax.axis_index("c")
        sid = lax.axis_index("s")
        wid = sid * _NC + cid

        zeros = jnp.zeros((16,), jnp.float32)

        def zrow(j, _):
            for i in range(_D // 16):
                zbuf[j, pl.ds(i * 16, 16)] = zeros
            return 0

        lax.fori_loop(0, _ZROWS, zrow, 0)

        base = _BASE_BLKS * wid + jnp.minimum(wid, _EXTRA)
        nblk = jnp.where(wid < _EXTRA, _BASE_BLKS + 1, _BASE_BLKS)

        def xstart(b):
            return jnp.minimum(_N_TC + (base + b) * _BLK, _N - _BLK)

        def start_io(b, slot):
            pltpu.async_copy(x_hbm.at[pl.ds(xstart(b), _BLK)],
                             xbuf.at[slot], gsem.at[slot])
            pltpu.async_copy(b_hbm.at[pl.ds(xstart(b), _BLK)],
                             idx2.at[slot], isem.at[slot])

        def wait_io(b, slot):
            pltpu.make_async_copy(x_hbm.at[pl.ds(xstart(b), _BLK)],
                                  xbuf.at[slot], gsem.at[slot]).wait()
            pltpu.make_async_copy(b_hbm.at[pl.ds(xstart(b), _BLK)],
                                  idx2.at[slot], isem.at[slot]).wait()

        start_io(0, 0)
        pltpu.sync_copy(zbuf, acc.at[pl.ds(sid * _ZROWS, _ZROWS)])
        plsc.subcore_barrier()

        def body(b, _):
            slot = lax.rem(b, 2)

            @pl.when(b < nblk)
            def _process():
                gstart = _N_TC + (base + b) * _BLK
                xs = xstart(b)
                wait_io(b, slot)

                @pl.when(xs != gstart)
                def _fixup():
                    for i in range(_BLK // 16):
                        r = xs + i * 16 + lax.iota(jnp.int32, 16)
                        v = idx2[slot, pl.ds(i * 16, 16)]
                        idx2[slot, pl.ds(i * 16, 16)] = (
                            jnp.where(r >= gstart, v, _G))

                @pl.when(b + 1 < nblk)
                def _prefetch():
                    nslot = lax.rem(b + 1, 2)

                    @pl.when(b >= 1)
                    def _drain_prev_scatter():
                        pltpu.make_async_copy(
                            xbuf.at[nslot], acc.at[idx2.at[nslot]],
                            ssem.at[nslot]).wait()

                    start_io(b + 1, nslot)

                pltpu.async_copy(xbuf.at[slot], acc.at[idx2.at[slot]],
                                 ssem.at[slot], add=True)

            return 0

        lax.fori_loop(0, _MAX_BLKS, body, 0)

        for s in range(2):
            pltpu.make_async_copy(xbuf.at[s], acc.at[idx2.at[s]],
                                  ssem.at[s]).wait()

        plsc.subcore_barrier()

        @pl.when(sid == 0)
        def _readout():
            pltpu.sync_copy(acc.at[pl.ds(0, _G)], out_hbm.at[cid])

    return seg_sum(x, batch)


def _tc_segment_sum(x, batch):
    def seg(b_ref, x_ref, o_ref):
        i = pl.program_id(0)

        @pl.when(i == 0)
        def _init():
            o_ref[...] = jnp.zeros_like(o_ref)

        gids = lax.broadcasted_iota(jnp.int32, (_G, _TC_BLK), 0)
        onehot = (b_ref[...] == gids).astype(jnp.bfloat16)
        o_ref[...] += jnp.dot(onehot, x_ref[...].astype(jnp.bfloat16),
                              preferred_element_type=jnp.float32)

    return pl.pallas_call(
        seg,
        grid=(_N_TC // _TC_BLK,),
        in_specs=[
            pl.BlockSpec((1, _TC_BLK), lambda i: (0, i)),
            pl.BlockSpec((_TC_BLK, _D), lambda i: (i, 0)),
        ],
        out_specs=pl.BlockSpec((_G, _D), lambda i: (0, 0)),
        out_shape=jax.ShapeDtypeStruct((_G, _D), jnp.float32),
        compiler_params=pltpu.CompilerParams(
            dimension_semantics=("arbitrary",)),
    )(batch.reshape(1, _N), x)


def _tc_mlp(sc_partials, tc_partial, W1, b1, W2p, b2):
    def mlp(p_ref, t_ref, w1_ref, b1_ref, w2_ref, b2_ref, o_ref):
        emb = p_ref[0] + p_ref[1] + t_ref[...]
        h = jnp.maximum(
            jnp.dot(emb, w1_ref[...], preferred_element_type=jnp.float32)
            + b1_ref[...], 0.0)
        o_ref[...] = (
            jnp.dot(h, w2_ref[...], preferred_element_type=jnp.float32)
            + b2_ref[...])

    return pl.pallas_call(
        mlp,
        out_shape=jax.ShapeDtypeStruct((_G, _D), jnp.float32),
    )(sc_partials, tc_partial, W1, b1, W2p, b2)


def kernel(x, batch, y, W1, b1, W2, b2):
    batch = batch.astype(jnp.int32)
    tc_partial = _tc_segment_sum(x, batch)
    sc_partials = _sc_segment_sum(x, batch)
    W2p = jnp.pad(W2, ((0, 0), (0, _D - W2.shape[1])))
    b2p = jnp.pad(b2, (0, _D - b2.shape[0]))
    out = _tc_mlp(sc_partials, tc_partial, W1, b1.reshape(1, _D),
                  W2p, b2p.reshape(1, _D))
    pred = out[:, : W2.shape[1]]
    return (pred, y)


# speedup vs baseline: 1.0210x; 1.0210x over previous
import functools

import jax
import jax.numpy as jnp
from jax import lax
from jax.experimental import pallas as pl
from jax.experimental.pallas import tpu as pltpu
from jax.experimental.pallas import tpu_sc as plsc

_G = 256
_N = 100000
_D = 128
_NC = 2
_NS = 16
_NW = _NC * _NS

_TC_BLK = 2048
_BAND = 64
_N_TC = 16 * _TC_BLK
_N_SC = _N - _N_TC

_BLK = 128
_SC_NBLKS = (_N_SC + _BLK - 1) // _BLK
_BASE_BLKS = _SC_NBLKS // _NW
_EXTRA = _SC_NBLKS - _BASE_BLKS * _NW
_MAX_BLKS = _BASE_BLKS + (1 if _EXTRA else 0)
_ZROWS = _G // _NS


def _sc_segment_sum(x, batch):
    mesh = plsc.VectorSubcoreMesh(core_axis_name="c", subcore_axis_name="s")

    @functools.partial(
        pl.kernel,
        out_type=jax.ShapeDtypeStruct((_NC, _G, _D), jnp.float32),
        mesh=mesh,
        scratch_types=[
            pltpu.VMEM((2, _BLK, _D), jnp.float32),
            pltpu.VMEM((2, _BLK), jnp.int32),
            pltpu.VMEM((_ZROWS, _D), jnp.float32),
            pltpu.VMEM_SHARED((_G + 8, _D), jnp.float32),
            pltpu.SemaphoreType.DMA((2,)),
            pltpu.SemaphoreType.DMA((2,)),
            pltpu.SemaphoreType.DMA((2,)),
        ],
    )
    def seg_sum(x_hbm, b_hbm, out_hbm, xbuf, idx2, zbuf, acc, gsem, isem,
                ssem):
        cid = lax.axis_index("c")
        sid = lax.axis_index("s")
        wid = sid * _NC + cid

        zeros = jnp.zeros((16,), jnp.float32)

        def zrow(j, _):
            for i in range(_D // 16):
                zbuf[j, pl.ds(i * 16, 16)] = zeros
            return 0

        lax.fori_loop(0, _ZROWS, zrow, 0)

        base = _BASE_BLKS * wid + jnp.minimum(wid, _EXTRA)
        nblk = jnp.where(wid < _EXTRA, _BASE_BLKS + 1, _BASE_BLKS)

        def xstart(b):
            return jnp.minimum(_N_TC + (base + b) * _BLK, _N - _BLK)

        def start_io(b, slot):
            pltpu.async_copy(x_hbm.at[pl.ds(xstart(b), _BLK)],
                             xbuf.at[slot], gsem.at[slot])
            pltpu.async_copy(b_hbm.at[pl.ds(xstart(b), _BLK)],
                             idx2.at[slot], isem.at[slot])

        def wait_io(b, slot):
            pltpu.make_async_copy(x_hbm.at[pl.ds(xstart(b), _BLK)],
                                  xbuf.at[slot], gsem.at[slot]).wait()
            pltpu.make_async_copy(b_hbm.at[pl.ds(xstart(b), _BLK)],
                                  idx2.at[slot], isem.at[slot]).wait()

        start_io(0, 0)
        pltpu.sync_copy(zbuf, acc.at[pl.ds(sid * _ZROWS, _ZROWS)])
        plsc.subcore_barrier()

        def body(b, _):
            slot = lax.rem(b, 2)

            @pl.when(b < nblk)
            def _process():
                gstart = _N_TC + (base + b) * _BLK
                xs = xstart(b)
                wait_io(b, slot)

                @pl.when(xs != gstart)
                def _fixup():
                    for i in range(_BLK // 16):
                        r = xs + i * 16 + lax.iota(jnp.int32, 16)
                        v = idx2[slot, pl.ds(i * 16, 16)]
                        idx2[slot, pl.ds(i * 16, 16)] = (
                            jnp.where(r >= gstart, v, _G))

                @pl.when(b + 1 < nblk)
                def _prefetch():
                    nslot = lax.rem(b + 1, 2)

                    @pl.when(b >= 1)
                    def _drain_prev_scatter():
                        pltpu.make_async_copy(
                            xbuf.at[nslot], acc.at[idx2.at[nslot]],
                            ssem.at[nslot]).wait()

                    start_io(b + 1, nslot)

                pltpu.async_copy(xbuf.at[slot], acc.at[idx2.at[slot]],
                                 ssem.at[slot], add=True)

            return 0

        lax.fori_loop(0, _MAX_BLKS, body, 0)

        for s in range(2):
            pltpu.make_async_copy(xbuf.at[s], acc.at[idx2.at[s]],
                                  ssem.at[s]).wait()

        plsc.subcore_barrier()

        @pl.when(sid == 0)
        def _readout():
            pltpu.sync_copy(acc.at[pl.ds(0, _G)], out_hbm.at[cid])

    return seg_sum(x, batch)


def _tc_segment_sum(x, batch):
    def seg(b_ref, x_ref, o_ref):
        i = pl.program_id(0)

        @pl.when(i == 0)
        def _init():
            o_ref[...] = jnp.zeros_like(o_ref)

        lo = b_ref[0, 0]
        hi = b_ref[0, _TC_BLK - 1]
        xb = x_ref[...].astype(jnp.bfloat16)
        for k in range(_G // _BAND):
            @pl.when(jnp.logical_and(hi >= k * _BAND,
                                     lo < (k + 1) * _BAND))
            def _band(k=k):
                gids = (lax.broadcasted_iota(jnp.int32, (_BAND, _TC_BLK), 0)
                        + k * _BAND)
                onehot = (b_ref[...] == gids).astype(jnp.bfloat16)
                o_ref[pl.ds(k * _BAND, _BAND)] += jnp.dot(
                    onehot, xb, preferred_element_type=jnp.float32)

    return pl.pallas_call(
        seg,
        grid=(_N_TC // _TC_BLK,),
        in_specs=[
            pl.BlockSpec((1, _TC_BLK), lambda i: (0, i)),
            pl.BlockSpec((_TC_BLK, _D), lambda i: (i, 0)),
        ],
        out_specs=pl.BlockSpec((_G, _D), lambda i: (0, 0)),
        out_shape=jax.ShapeDtypeStruct((_G, _D), jnp.float32),
        compiler_params=pltpu.CompilerParams(
            dimension_semantics=("arbitrary",)),
    )(batch.reshape(1, _N), x)


def _tc_mlp(sc_partials, tc_partial, W1, b1, W2p, b2):
    def mlp(p_ref, t_ref, w1_ref, b1_ref, w2_ref, b2_ref, o_ref):
        emb = p_ref[0] + p_ref[1] + t_ref[...]
        h = jnp.maximum(
            jnp.dot(emb, w1_ref[...], preferred_element_type=jnp.float32)
            + b1_ref[...], 0.0)
        o_ref[...] = (
            jnp.dot(h, w2_ref[...], preferred_element_type=jnp.float32)
            + b2_ref[...])

    return pl.pallas_call(
        mlp,
        out_shape=jax.ShapeDtypeStruct((_G, _D), jnp.float32),
    )(sc_partials, tc_partial, W1, b1, W2p, b2)


def kernel(x, batch, y, W1, b1, W2, b2):
    batch = batch.astype(jnp.int32)
    tc_partial = _tc_segment_sum(x, batch)
    sc_partials = _sc_segment_sum(x, batch)
    W2p = jnp.pad(W2, ((0, 0), (0, _D - W2.shape[1])))
    b2p = jnp.pad(b2, (0, _D - b2.shape[0]))
    out = _tc_mlp(sc_partials, tc_partial, W1, b1.reshape(1, _D),
                  W2p, b2p.reshape(1, _D))
    pred = out[:, : W2.shape[1]]
    return (pred, y)


# speedup vs baseline: 1.0678x; 1.0459x over previous
import functools

import jax
import jax.numpy as jnp
from jax import lax
from jax.experimental import pallas as pl
from jax.experimental.pallas import tpu as pltpu
from jax.experimental.pallas import tpu_sc as plsc

_G = 256
_N = 100000
_D = 128
_NC = 2
_NS = 16
_NW = _NC * _NS

_TC_BLK = 2048
_BAND = 64
_N_TC = 20 * _TC_BLK
_N_SC = _N - _N_TC

_BLK = 128
_SC_NBLKS = (_N_SC + _BLK - 1) // _BLK
_BASE_BLKS = _SC_NBLKS // _NW
_EXTRA = _SC_NBLKS - _BASE_BLKS * _NW
_MAX_BLKS = _BASE_BLKS + (1 if _EXTRA else 0)
_ZROWS = _G // _NS


def _sc_segment_sum(x, batch):
    mesh = plsc.VectorSubcoreMesh(core_axis_name="c", subcore_axis_name="s")

    @functools.partial(
        pl.kernel,
        out_type=jax.ShapeDtypeStruct((_NC, _G, _D), jnp.float32),
        mesh=mesh,
        scratch_types=[
            pltpu.VMEM((2, _BLK, _D), jnp.float32),
            pltpu.VMEM((2, _BLK), jnp.int32),
            pltpu.VMEM((_ZROWS, _D), jnp.float32),
            pltpu.VMEM_SHARED((_G + 8, _D), jnp.float32),
            pltpu.SemaphoreType.DMA((2,)),
            pltpu.SemaphoreType.DMA((2,)),
            pltpu.SemaphoreType.DMA((2,)),
        ],
    )
    def seg_sum(x_hbm, b_hbm, out_hbm, xbuf, idx2, zbuf, acc, gsem, isem,
                ssem):
        cid = lax.axis_index("c")
        sid = lax.axis_index("s")
        wid = sid * _NC + cid

        zeros = jnp.zeros((16,), jnp.float32)

        def zrow(j, _):
            for i in range(_D // 16):
                zbuf[j, pl.ds(i * 16, 16)] = zeros
            return 0

        lax.fori_loop(0, _ZROWS, zrow, 0)

        base = _BASE_BLKS * wid + jnp.minimum(wid, _EXTRA)
        nblk = jnp.where(wid < _EXTRA, _BASE_BLKS + 1, _BASE_BLKS)

        def xstart(b):
            return jnp.minimum(_N_TC + (base + b) * _BLK, _N - _BLK)

        def start_io(b, slot):
            pltpu.async_copy(x_hbm.at[pl.ds(xstart(b), _BLK)],
                             xbuf.at[slot], gsem.at[slot])
            pltpu.async_copy(b_hbm.at[pl.ds(xstart(b), _BLK)],
                             idx2.at[slot], isem.at[slot])

        def wait_io(b, slot):
            pltpu.make_async_copy(x_hbm.at[pl.ds(xstart(b), _BLK)],
                                  xbuf.at[slot], gsem.at[slot]).wait()
            pltpu.make_async_copy(b_hbm.at[pl.ds(xstart(b), _BLK)],
                                  idx2.at[slot], isem.at[slot]).wait()

        start_io(0, 0)
        pltpu.sync_copy(zbuf, acc.at[pl.ds(sid * _ZROWS, _ZROWS)])
        plsc.subcore_barrier()

        def body(b, _):
            slot = lax.rem(b, 2)

            @pl.when(b < nblk)
            def _process():
                gstart = _N_TC + (base + b) * _BLK
                xs = xstart(b)
                wait_io(b, slot)

                @pl.when(xs != gstart)
                def _fixup():
                    for i in range(_BLK // 16):
                        r = xs + i * 16 + lax.iota(jnp.int32, 16)
                        v = idx2[slot, pl.ds(i * 16, 16)]
                        idx2[slot, pl.ds(i * 16, 16)] = (
                            jnp.where(r >= gstart, v, _G))

                @pl.when(b + 1 < nblk)
                def _prefetch():
                    nslot = lax.rem(b + 1, 2)

                    @pl.when(b >= 1)
                    def _drain_prev_scatter():
                        pltpu.make_async_copy(
                            xbuf.at[nslot], acc.at[idx2.at[nslot]],
                            ssem.at[nslot]).wait()

                    start_io(b + 1, nslot)

                pltpu.async_copy(xbuf.at[slot], acc.at[idx2.at[slot]],
                                 ssem.at[slot], add=True)

            return 0

        lax.fori_loop(0, _MAX_BLKS, body, 0)

        for s in range(2):
            pltpu.make_async_copy(xbuf.at[s], acc.at[idx2.at[s]],
                                  ssem.at[s]).wait()

        plsc.subcore_barrier()

        @pl.when(sid == 0)
        def _readout():
            pltpu.sync_copy(acc.at[pl.ds(0, _G)], out_hbm.at[cid])

    return seg_sum(x, batch)


def _tc_segment_sum(x, batch):
    def seg(b_ref, x_ref, o_ref):
        i = pl.program_id(0)

        @pl.when(i == 0)
        def _init():
            o_ref[...] = jnp.zeros_like(o_ref)

        lo = b_ref[0, 0]
        hi = b_ref[0, _TC_BLK - 1]
        xb = x_ref[...].astype(jnp.bfloat16)
        for k in range(_G // _BAND):
            @pl.when(jnp.logical_and(hi >= k * _BAND,
                                     lo < (k + 1) * _BAND))
            def _band(k=k):
                gids = (lax.broadcasted_iota(jnp.int32, (_BAND, _TC_BLK), 0)
                        + k * _BAND)
                onehot = (b_ref[...] == gids).astype(jnp.bfloat16)
                o_ref[pl.ds(k * _BAND, _BAND)] += jnp.dot(
                    onehot, xb, preferred_element_type=jnp.float32)

    return pl.pallas_call(
        seg,
        grid=(_N_TC // _TC_BLK,),
        in_specs=[
            pl.BlockSpec((1, _TC_BLK), lambda i: (0, i)),
            pl.BlockSpec((_TC_BLK, _D), lambda i: (i, 0)),
        ],
        out_specs=pl.BlockSpec((_G, _D), lambda i: (0, 0)),
        out_shape=jax.ShapeDtypeStruct((_G, _D), jnp.float32),
        compiler_params=pltpu.CompilerParams(
            dimension_semantics=("arbitrary",)),
    )(batch.reshape(1, _N), x)


def _tc_mlp(sc_partials, tc_partial, W1, b1, W2p, b2):
    def mlp(p_ref, t_ref, w1_ref, b1_ref, w2_ref, b2_ref, o_ref):
        emb = p_ref[0] + p_ref[1] + t_ref[...]
        h = jnp.maximum(
            jnp.dot(emb, w1_ref[...], preferred_element_type=jnp.float32)
            + b1_ref[...], 0.0)
        o_ref[...] = (
            jnp.dot(h, w2_ref[...], preferred_element_type=jnp.float32)
            + b2_ref[...])

    return pl.pallas_call(
        mlp,
        out_shape=jax.ShapeDtypeStruct((_G, _D), jnp.float32),
    )(sc_partials, tc_partial, W1, b1, W2p, b2)


def kernel(x, batch, y, W1, b1, W2, b2):
    batch = batch.astype(jnp.int32)
    tc_partial = _tc_segment_sum(x, batch)
    sc_partials = _sc_segment_sum(x, batch)
    W2p = jnp.pad(W2, ((0, 0), (0, _D - W2.shape[1])))
    b2p = jnp.pad(b2, (0, _D - b2.shape[0]))
    out = _tc_mlp(sc_partials, tc_partial, W1, b1.reshape(1, _D),
                  W2p, b2p.reshape(1, _D))
    pred = out[:, : W2.shape[1]]
    return (pred, y)


# speedup vs baseline: 1.0763x; 1.0080x over previous
import functools

import jax
import jax.numpy as jnp
from jax import lax
from jax.experimental import pallas as pl
from jax.experimental.pallas import tpu as pltpu
from jax.experimental.pallas import tpu_sc as plsc

_G = 256
_N = 100000
_D = 128
_NC = 2
_NS = 16
_NW = _NC * _NS

_TC_BLK = 2048
_BAND = 64
_N_TC = 24 * _TC_BLK
_N_SC = _N - _N_TC

_BLK = 128
_SC_NBLKS = (_N_SC + _BLK - 1) // _BLK
_BASE_BLKS = _SC_NBLKS // _NW
_EXTRA = _SC_NBLKS - _BASE_BLKS * _NW
_MAX_BLKS = _BASE_BLKS + (1 if _EXTRA else 0)
_ZROWS = _G // _NS


def _sc_segment_sum(x, batch):
    mesh = plsc.VectorSubcoreMesh(core_axis_name="c", subcore_axis_name="s")

    @functools.partial(
        pl.kernel,
        out_type=jax.ShapeDtypeStruct((_NC, _G, _D), jnp.float32),
        mesh=mesh,
        scratch_types=[
            pltpu.VMEM((2, _BLK, _D), jnp.float32),
            pltpu.VMEM((2, _BLK), jnp.int32),
            pltpu.VMEM((_ZROWS, _D), jnp.float32),
            pltpu.VMEM_SHARED((_G + 8, _D), jnp.float32),
            pltpu.SemaphoreType.DMA((2,)),
            pltpu.SemaphoreType.DMA((2,)),
            pltpu.SemaphoreType.DMA((2,)),
        ],
    )
    def seg_sum(x_hbm, b_hbm, out_hbm, xbuf, idx2, zbuf, acc, gsem, isem,
                ssem):
        cid = lax.axis_index("c")
        sid = lax.axis_index("s")
        wid = sid * _NC + cid

        zeros = jnp.zeros((16,), jnp.float32)

        def zrow(j, _):
            for i in range(_D // 16):
                zbuf[j, pl.ds(i * 16, 16)] = zeros
            return 0

        lax.fori_loop(0, _ZROWS, zrow, 0)

        base = _BASE_BLKS * wid + jnp.minimum(wid, _EXTRA)
        nblk = jnp.where(wid < _EXTRA, _BASE_BLKS + 1, _BASE_BLKS)

        def xstart(b):
            return jnp.minimum(_N_TC + (base + b) * _BLK, _N - _BLK)

        def start_io(b, slot):
            pltpu.async_copy(x_hbm.at[pl.ds(xstart(b), _BLK)],
                             xbuf.at[slot], gsem.at[slot])
            pltpu.async_copy(b_hbm.at[pl.ds(xstart(b), _BLK)],
                             idx2.at[slot], isem.at[slot])

        def wait_io(b, slot):
            pltpu.make_async_copy(x_hbm.at[pl.ds(xstart(b), _BLK)],
                                  xbuf.at[slot], gsem.at[slot]).wait()
            pltpu.make_async_copy(b_hbm.at[pl.ds(xstart(b), _BLK)],
                                  idx2.at[slot], isem.at[slot]).wait()

        start_io(0, 0)
        pltpu.sync_copy(zbuf, acc.at[pl.ds(sid * _ZROWS, _ZROWS)])
        plsc.subcore_barrier()

        def body(b, _):
            slot = lax.rem(b, 2)

            @pl.when(b < nblk)
            def _process():
                gstart = _N_TC + (base + b) * _BLK
                xs = xstart(b)
                wait_io(b, slot)

                @pl.when(xs != gstart)
                def _fixup():
                    for i in range(_BLK // 16):
                        r = xs + i * 16 + lax.iota(jnp.int32, 16)
                        v = idx2[slot, pl.ds(i * 16, 16)]
                        idx2[slot, pl.ds(i * 16, 16)] = (
                            jnp.where(r >= gstart, v, _G))

                @pl.when(b + 1 < nblk)
                def _prefetch():
                    nslot = lax.rem(b + 1, 2)

                    @pl.when(b >= 1)
                    def _drain_prev_scatter():
                        pltpu.make_async_copy(
                            xbuf.at[nslot], acc.at[idx2.at[nslot]],
                            ssem.at[nslot]).wait()

                    start_io(b + 1, nslot)

                pltpu.async_copy(xbuf.at[slot], acc.at[idx2.at[slot]],
                                 ssem.at[slot], add=True)

            return 0

        lax.fori_loop(0, _MAX_BLKS, body, 0)

        for s in range(2):
            pltpu.make_async_copy(xbuf.at[s], acc.at[idx2.at[s]],
                                  ssem.at[s]).wait()

        plsc.subcore_barrier()

        @pl.when(sid == 0)
        def _readout():
            pltpu.sync_copy(acc.at[pl.ds(0, _G)], out_hbm.at[cid])

    return seg_sum(x, batch)


def _tc_segment_sum(x, batch):
    def seg(b_ref, x_ref, o_ref):
        i = pl.program_id(0)

        @pl.when(i == 0)
        def _init():
            o_ref[...] = jnp.zeros_like(o_ref)

        lo = b_ref[0, 0]
        hi = b_ref[0, _TC_BLK - 1]
        xb = x_ref[...].astype(jnp.bfloat16)
        for k in range(_G // _BAND):
            @pl.when(jnp.logical_and(hi >= k * _BAND,
                                     lo < (k + 1) * _BAND))
            def _band(k=k):
                gids = (lax.broadcasted_iota(jnp.int32, (_BAND, _TC_BLK), 0)
                        + k * _BAND)
                onehot = (b_ref[...] == gids).astype(jnp.bfloat16)
                o_ref[pl.ds(k * _BAND, _BAND)] += jnp.dot(
                    onehot, xb, preferred_element_type=jnp.float32)

    return pl.pallas_call(
        seg,
        grid=(_N_TC // _TC_BLK,),
        in_specs=[
            pl.BlockSpec((1, _TC_BLK), lambda i: (0, i)),
            pl.BlockSpec((_TC_BLK, _D), lambda i: (i, 0)),
        ],
        out_specs=pl.BlockSpec((_G, _D), lambda i: (0, 0)),
        out_shape=jax.ShapeDtypeStruct((_G, _D), jnp.float32),
        compiler_params=pltpu.CompilerParams(
            dimension_semantics=("arbitrary",)),
    )(batch.reshape(1, _N), x)


def _tc_mlp(sc_partials, tc_partial, W1, b1, W2p, b2):
    def mlp(p_ref, t_ref, w1_ref, b1_ref, w2_ref, b2_ref, o_ref):
        emb = p_ref[0] + p_ref[1] + t_ref[...]
        h = jnp.maximum(
            jnp.dot(emb, w1_ref[...], preferred_element_type=jnp.float32)
            + b1_ref[...], 0.0)
        o_ref[...] = (
            jnp.dot(h, w2_ref[...], preferred_element_type=jnp.float32)
            + b2_ref[...])

    return pl.pallas_call(
        mlp,
        out_shape=jax.ShapeDtypeStruct((_G, _D), jnp.float32),
    )(sc_partials, tc_partial, W1, b1, W2p, b2)


def kernel(x, batch, y, W1, b1, W2, b2):
    batch = batch.astype(jnp.int32)
    tc_partial = _tc_segment_sum(x, batch)
    sc_partials = _sc_segment_sum(x, batch)
    W2p = jnp.pad(W2, ((0, 0), (0, _D - W2.shape[1])))
    b2p = jnp.pad(b2, (0, _D - b2.shape[0]))
    out = _tc_mlp(sc_partials, tc_partial, W1, b1.reshape(1, _D),
                  W2p, b2p.reshape(1, _D))
    pred = out[:, : W2.shape[1]]
    return (pred, y)
